# baseline (speedup 1.0000x reference)
import functools

import jax
import jax.numpy as jnp
from jax.experimental import pallas as pl
from jax.experimental.pallas import tpu as pltpu
from jax.experimental.pallas import tpu_sc as plsc

_TOP_K = 5


def _top5_lanes(v, axis):
    iota = jax.lax.broadcasted_iota(jnp.int32, v.shape, axis)
    x = v
    ms = []
    for _ in range(_TOP_K):
        m = jnp.max(x, axis=axis, keepdims=True)
        ms.append(m)
        idx = jnp.min(jnp.where(x == m, iota, 2**30), axis=axis,
                      keepdims=True)
        x = jnp.where(iota == idx, -jnp.inf, x)
    return ms


def _fused_kernel(inv_hw, x_ref, d_ref, o_ref, sd_ref):
    C = x_ref.shape[1]
    x = x_ref[0]
    e = jnp.exp(x)
    tot = jnp.sum(e, axis=0)
    recip = 1.0 / tot
    sd = jnp.sum(e * recip, axis=(1, 2), keepdims=True) * inv_hw

    sdl = sd.reshape(1, 1, C)
    sd_ref[...] = sdl
    dist = d_ref[...]
    thresh = _top5_lanes(sdl, 2)[-1]
    mask = (sdl > thresh).astype(jnp.float32)
    rl = (dist - sdl) * mask
    r = rl.reshape(C, 1, 1)

    o_ref[0] = e * recip + r


_CPAD = 160
_NCHUNK = _CPAD // 16
_PADVAL = -1e30


def _sc_shuffle(x, idx):
    dnums = jax.lax.GatherDimensionNumbers(
        offset_dims=(), collapsed_slice_dims=(0,), start_index_map=(0,))
    return jax.lax.gather(
        x, idx[:, None], dnums, (1,),
        mode=jax.lax.GatherScatterMode.PROMISE_IN_BOUNDS)


def _sc_allreduce(x, lane, op):
    for sh in (8, 4, 2, 1):
        x = op(x, _sc_shuffle(x, jnp.bitwise_xor(lane, sh)))
    return x


def _sc_extract5(v_ref, lane):
    tops = []
    for _ in range(_TOP_K):
        m = v_ref[pl.ds(0, 16)]
        for j in range(1, _NCHUNK):
            m = jnp.maximum(m, v_ref[pl.ds(16 * j, 16)])
        s = _sc_allreduce(m, lane, jnp.maximum)
        cand = jnp.full((16,), 2**30, jnp.int32)
        for j in range(_NCHUNK):
            xj = v_ref[pl.ds(16 * j, 16)]
            cand = jnp.minimum(
                cand, jnp.where(xj == s, lane + 16 * j, 2**30))
        g = _sc_allreduce(cand, lane, jnp.minimum)
        for j in range(_NCHUNK):
            xj = v_ref[pl.ds(16 * j, 16)]
            v_ref[pl.ds(16 * j, 16)] = jnp.where(
                lane + 16 * j == g, _PADVAL, xj)
        tops.append(s)
    return tops


def _sc_topk_body(sd_hbm, d_hbm, ksm_hbm, klab_hbm, kalt_hbm,
                  work_v, orig_v, d_v, b_v, out_v):
    n_rows = sd_hbm.shape[0]
    wid = jax.lax.axis_index("s") * 2 + jax.lax.axis_index("c")

    @pl.when(wid < n_rows)
    def _():
        row = wid
        pltpu.sync_copy(sd_hbm.at[row], work_v)
        pltpu.sync_copy(sd_hbm.at[row], orig_v)
        pltpu.sync_copy(d_hbm.at[row], d_v)

        lane = jax.lax.iota(jnp.int32, 16)
        tops_sd = _sc_extract5(work_v, lane)
        t = tops_sd[-1]

        acc_a = jnp.zeros((16,), jnp.float32)
        acc_b = jnp.zeros((16,), jnp.float32)
        for j in range(_NCHUNK):
            oj = orig_v[pl.ds(16 * j, 16)]
            dj = d_v[pl.ds(16 * j, 16)]
            mj = oj > t
            acc_a = acc_a + jnp.where(mj, oj, 0.0)
            bj = jnp.where(mj, dj, 0.0)
            b_v[pl.ds(16 * j, 16)] = bj
            acc_b = acc_b + bj
        sum_a = _sc_allreduce(acc_a, lane, jnp.add)
        sum_b = _sc_allreduce(acc_b, lane, jnp.add)

        tops_b = _sc_extract5(b_v, lane)

        zero16 = jnp.zeros((16,), jnp.float32)
        ksm = zero16
        klab = zero16
        kalt = zero16
        for i in range(_TOP_K):
            a_i = jnp.where(tops_sd[i] > t, tops_sd[i], 0.0)
            ksm = jnp.where(lane == i, a_i / sum_a, ksm)
            klab = jnp.where(lane == i, tops_b[i] / (sum_b + 1e-12), klab)
            kalt = jnp.where(lane == i, tops_b[i] / sum_b, kalt)

        out_v[...] = ksm
        pltpu.sync_copy(out_v, ksm_hbm.at[row])
        out_v[...] = klab
        pltpu.sync_copy(out_v, klab_hbm.at[row])
        out_v[...] = kalt
        pltpu.sync_copy(out_v, kalt_hbm.at[row])


def kernel(logits, distribution):
    N, C, H, W = logits.shape

    dist_l = distribution.reshape(N, 1, C)
    corrected, sd = pl.pallas_call(
        functools.partial(_fused_kernel, 1.0 / (H * W)),
        grid=(N,),
        in_specs=[
            pl.BlockSpec((1, C, H, W), lambda n: (n, 0, 0, 0)),
            pl.BlockSpec((1, 1, C), lambda n: (n, 0, 0)),
        ],
        out_specs=[
            pl.BlockSpec((1, C, H, W), lambda n: (n, 0, 0, 0)),
            pl.BlockSpec((1, 1, C), lambda n: (n, 0, 0)),
        ],
        out_shape=[
            jax.ShapeDtypeStruct((N, C, H, W), jnp.float32),
            jax.ShapeDtypeStruct((N, 1, C), jnp.float32),
        ],
        compiler_params=pltpu.CompilerParams(
            dimension_semantics=("parallel",)),
    )(logits, dist_l)

    sd_pad = jnp.pad(sd.reshape(N, C), ((0, 0), (0, _CPAD - C)),
                     constant_values=_PADVAL)
    d_pad = jnp.pad(distribution.reshape(N, C), ((0, 0), (0, _CPAD - C)))

    sc_topk = functools.partial(
        pl.kernel,
        out_type=[
            jax.ShapeDtypeStruct((N, 16), jnp.float32),
            jax.ShapeDtypeStruct((N, 16), jnp.float32),
            jax.ShapeDtypeStruct((N, 16), jnp.float32),
        ],
        scratch_types=[
            pltpu.VMEM((_CPAD,), jnp.float32),
            pltpu.VMEM((_CPAD,), jnp.float32),
            pltpu.VMEM((_CPAD,), jnp.float32),
            pltpu.VMEM((_CPAD,), jnp.float32),
            pltpu.VMEM((16,), jnp.float32),
        ],
        mesh=plsc.VectorSubcoreMesh(core_axis_name="c", subcore_axis_name="s"),
    )(_sc_topk_body)
    ksm, klab, kalt = sc_topk(sd_pad, d_pad)

    k1 = klab[:, :_TOP_K].reshape(N, _TOP_K, 1, 1)
    k2 = ksm[:, :_TOP_K].reshape(N, _TOP_K, 1, 1)
    k3 = kalt[:, :_TOP_K].reshape(N, _TOP_K, 1, 1)
    return (corrected, k1, k2, k3)
